# baseline (speedup 1.0000x reference)
import functools

import jax
import jax.numpy as jnp
from jax import lax
from jax.experimental import pallas as pl
from jax.experimental.pallas import tpu as pltpu
from jax.experimental.pallas import tpu_sc as plsc

_SAMPLES = 4
_TIME = 4096
_DIM = 32
_WORDS = _TIME * _DIM


@functools.lru_cache(maxsize=None)
def _build(samples: int, words: int):
    info = plsc.get_sparse_core_info()
    nw = info.num_cores * info.num_subcores
    assert words % nw == 0
    chunk = words // nw

    mesh = plsc.VectorSubcoreMesh(core_axis_name="c", subcore_axis_name="s")

    @functools.partial(
        pl.kernel,
        out_type=jax.ShapeDtypeStruct((samples * words,), jnp.float32),
        mesh=mesh,
        scratch_types=[
            pltpu.VMEM((chunk,), jnp.float32),
            pltpu.SemaphoreType.DMA,
        ],
    )
    def tile_kernel(table_hbm, out_hbm, buf, sem):
        wid = lax.axis_index("s") * info.num_cores + lax.axis_index("c")
        base = wid * chunk
        pltpu.sync_copy(table_hbm.at[pl.ds(base, chunk)], buf)
        copies = [
            pltpu.async_copy(buf, out_hbm.at[pl.ds(s * words + base, chunk)], sem)
            for s in range(samples)
        ]
        for c in copies:
            c.wait()

    return tile_kernel


def kernel(x, table):
    samples = x.shape[0]
    flat = _build(samples, table.size)(table.reshape(-1))
    return flat.reshape(samples, table.shape[0], table.shape[1])


# speedup vs baseline: 1.1270x; 1.1270x over previous
import functools

import jax
import jax.numpy as jnp
from jax import lax
from jax.experimental import pallas as pl
from jax.experimental.pallas import tpu as pltpu
from jax.experimental.pallas import tpu_sc as plsc


@functools.lru_cache(maxsize=None)
def _build(samples: int, time: int, dim: int):
    info = plsc.get_sparse_core_info()
    nw = info.num_cores * info.num_subcores
    assert time % nw == 0
    rows = time // nw

    mesh = plsc.VectorSubcoreMesh(core_axis_name="c", subcore_axis_name="s")

    @functools.partial(
        pl.kernel,
        out_type=jax.ShapeDtypeStruct((samples, time, dim), jnp.float32),
        mesh=mesh,
        scratch_types=[
            pltpu.VMEM((rows, dim), jnp.float32),
            pltpu.SemaphoreType.DMA,
        ],
        compiler_params=pltpu.CompilerParams(use_tc_tiling_on_sc=True),
    )
    def tile_kernel(table_hbm, out_hbm, buf, sem):
        wid = lax.axis_index("s") * info.num_cores + lax.axis_index("c")
        base = wid * rows
        pltpu.sync_copy(table_hbm.at[pl.ds(base, rows), :], buf)
        copies = [
            pltpu.async_copy(buf, out_hbm.at[s, pl.ds(base, rows), :], sem)
            for s in range(samples)
        ]
        for c in copies:
            c.wait()

    return tile_kernel


def kernel(x, table):
    return _build(x.shape[0], table.shape[0], table.shape[1])(table)


# speedup vs baseline: 1.6285x; 1.4450x over previous
import functools

import jax
import jax.numpy as jnp
from jax import lax
from jax.experimental import pallas as pl
from jax.experimental.pallas import tpu as pltpu
from jax.experimental.pallas import tpu_sc as plsc


@functools.lru_cache(maxsize=None)
def _build(samples: int, time: int, dim: int):
    info = plsc.get_sparse_core_info()
    nw = info.num_cores * info.num_subcores
    sub_blocks = dim // 8
    lane_chunks = nw // sub_blocks
    assert dim % 8 == 0 and time % (128 * lane_chunks) == 0
    cols = time // lane_chunks

    mesh = plsc.VectorSubcoreMesh(core_axis_name="c", subcore_axis_name="s")

    @functools.partial(
        pl.kernel,
        out_type=jax.ShapeDtypeStruct((samples, dim, time), jnp.float32),
        mesh=mesh,
        scratch_types=[
            pltpu.VMEM((8, cols), jnp.float32),
            pltpu.SemaphoreType.DMA,
        ],
        compiler_params=pltpu.CompilerParams(use_tc_tiling_on_sc=True),
    )
    def tile_kernel(table_hbm, out_hbm, buf, sem):
        wid = lax.axis_index("s") * info.num_cores + lax.axis_index("c")
        row = (wid // lane_chunks) * 8
        col = (wid % lane_chunks) * cols
        pltpu.sync_copy(table_hbm.at[pl.ds(row, 8), pl.ds(col, cols)], buf)
        copies = [
            pltpu.async_copy(
                buf, out_hbm.at[s, pl.ds(row, 8), pl.ds(col, cols)], sem
            )
            for s in range(samples)
        ]
        for c in copies:
            c.wait()

    return tile_kernel


def kernel(x, table):
    table_t = jnp.swapaxes(table, 0, 1)
    out_t = _build(x.shape[0], table.shape[0], table.shape[1])(table_t)
    return jnp.swapaxes(out_t, 1, 2)
